# baseline (speedup 1.0000x reference)
import functools

import jax
import jax.numpy as jnp
import numpy as np
from jax.experimental import pallas as pl
from jax.experimental.pallas import tpu as pltpu


def _dft_mats(C: int, FP: int):
    F = C // 2 + 1
    c = np.arange(C)[:, None].astype(np.float64)
    f = np.arange(FP)[None, :].astype(np.float64)
    ang = 2.0 * np.pi * c * f / C
    valid = (f < F).astype(np.float64)
    cosm = np.cos(ang) * valid
    sinm = -np.sin(ang) * valid
    fwd = np.concatenate([cosm, sinm], axis=1)
    w = np.full((FP, 1), 2.0 / C)
    w[0, 0] = 1.0 / C
    if F - 1 < FP:
        w[F - 1, 0] = 1.0 / C
    angT = 2.0 * np.pi * np.arange(FP)[:, None].astype(np.float64) * np.arange(C)[None, :] / C
    validT = (np.arange(FP)[:, None] < F).astype(np.float64)
    icos = np.cos(angT) * w * validT
    isin = -np.sin(angT) * w * validT
    inv = np.concatenate([icos, isin], axis=0)
    return fwd.astype(np.float32), inv.astype(np.float32)


def _fan_block(x_ref, fwd_ref, inv_ref, w1_ref, b1_ref, w2_ref, b2_ref,
               w3_ref, b3_ref, o_ref, *, F: int, FP: int, K: int):
    x = x_ref[...]
    hi = jax.lax.Precision.HIGHEST
    spec = jnp.dot(x, fwd_ref[...], preferred_element_type=jnp.float32,
                   precision=hi)
    re = spec[:, :FP]
    im = spec[:, FP:]
    mag = re * re + im * im
    cols = jax.lax.broadcasted_iota(jnp.int32, mag.shape, 1)
    neg_inf = jnp.float32(-jnp.inf)
    mag = jnp.where(cols < F, mag, neg_inf)

    def body(_, m):
        mx = jnp.max(m, axis=1, keepdims=True)
        return jnp.where(m == mx, neg_inf, m)

    m_fin = jax.lax.fori_loop(0, K, body, mag)
    keep = jnp.where((m_fin == neg_inf) & (cols < F),
                     jnp.float32(1.0), jnp.float32(0.0))

    lo = jax.lax.Precision.DEFAULT
    spec_m = spec * jnp.concatenate([keep, keep], axis=1)
    x_filt = jnp.dot(spec_m, inv_ref[...],
                     preferred_element_type=jnp.float32, precision=lo)
    pf = jnp.maximum(
        jnp.dot(x_filt, w1_ref[...], preferred_element_type=jnp.float32,
                precision=lo) + b1_ref[...], 0.0)
    comb = jnp.concatenate([pf, x], axis=1)
    h = jnp.maximum(
        jnp.dot(comb, w2_ref[...], preferred_element_type=jnp.float32,
                precision=lo) + b2_ref[...], 0.0)
    out_mlp = jnp.dot(h, w3_ref[...], preferred_element_type=jnp.float32,
                      precision=lo) + b3_ref[...]
    o_ref[...] = (x - x_filt) + out_mlp


@jax.jit
def kernel(x, W1, b1, W2, b2, W3, b3):
    B, S, C = x.shape
    F = C // 2 + 1
    FP = ((F + 127) // 128) * 128
    K = min(20, F)
    T = B * S
    TB = 1024 if T % 1024 == 0 else T

    fwd_np, inv_np = _dft_mats(C, FP)
    fwd = jnp.asarray(fwd_np)
    inv = jnp.asarray(inv_np)

    H1 = W1.shape[1]
    H1P = 128
    w1p = jnp.zeros((C, H1P), jnp.float32).at[:, :H1].set(W1)
    b1p = jnp.zeros((1, H1P), jnp.float32).at[0, :H1].set(b1)
    H2 = W2.shape[1]
    w2p = jnp.zeros((H1P + C, H2), jnp.float32)
    w2p = w2p.at[:H1, :].set(W2[:H1, :]).at[H1P:, :].set(W2[H1:, :])
    b2r = b2.reshape(1, H2)
    b3r = b3.reshape(1, C)

    xt = x.reshape(T, C)
    full = lambda shape: pl.BlockSpec(shape, lambda i: (0, 0))
    out = pl.pallas_call(
        functools.partial(_fan_block, F=F, FP=FP, K=K),
        grid=(T // TB,),
        in_specs=[
            pl.BlockSpec((TB, C), lambda i: (i, 0)),
            full((C, 2 * FP)),
            full((2 * FP, C)),
            full((C, H1P)),
            full((1, H1P)),
            full((H1P + C, H2)),
            full((1, H2)),
            full((H2, C)),
            full((1, C)),
        ],
        out_specs=pl.BlockSpec((TB, C), lambda i: (i, 0)),
        out_shape=jax.ShapeDtypeStruct((T, C), jnp.float32),
        compiler_params=pltpu.CompilerParams(
            dimension_semantics=("parallel",)),
    )(xt, fwd, inv, w1p, b1p, w2p, b2r, W3, b3r)
    return out.reshape(B, S, C)


# speedup vs baseline: 1.2390x; 1.2390x over previous
import functools

import jax
import jax.numpy as jnp
import numpy as np
from jax.experimental import pallas as pl
from jax.experimental.pallas import tpu as pltpu


def _dft_mats(C: int, FP: int):
    F = C // 2 + 1
    c = np.arange(C)[:, None].astype(np.float64)
    f = np.arange(FP)[None, :].astype(np.float64)
    ang = 2.0 * np.pi * c * f / C
    valid = (f < F).astype(np.float64)
    cosm = np.cos(ang) * valid
    sinm = -np.sin(ang) * valid
    fwd = np.concatenate([cosm, sinm], axis=1)
    w = np.full((FP, 1), 2.0 / C)
    w[0, 0] = 1.0 / C
    if F - 1 < FP:
        w[F - 1, 0] = 1.0 / C
    angT = 2.0 * np.pi * np.arange(FP)[:, None].astype(np.float64) * np.arange(C)[None, :] / C
    validT = (np.arange(FP)[:, None] < F).astype(np.float64)
    icos = np.cos(angT) * w * validT
    isin = -np.sin(angT) * w * validT
    inv = np.concatenate([icos, isin], axis=0)
    return fwd.astype(np.float32), inv.astype(np.float32)


def _fan_block(x_ref, fwdh_ref, fwdl_ref, inv_ref, w1_ref, b1_ref, w2_ref,
               b2_ref, w3_ref, b3_ref, o_ref, *, F: int, FP: int, K: int):
    x = x_ref[...]
    xh = x.astype(jnp.bfloat16)
    xl = (x - xh.astype(jnp.float32)).astype(jnp.bfloat16)
    fh = fwdh_ref[...]
    spec = (jnp.dot(xh, fh, preferred_element_type=jnp.float32)
            + (jnp.dot(xh, fwdl_ref[...], preferred_element_type=jnp.float32)
               + jnp.dot(xl, fh, preferred_element_type=jnp.float32)))
    re = spec[:, :FP]
    im = spec[:, FP:]
    mag = re * re + im * im
    cols = jax.lax.broadcasted_iota(jnp.int32, mag.shape, 1)
    neg_inf = jnp.float32(-jnp.inf)
    mag = jnp.where(cols < F, mag, neg_inf)

    def body(_, m):
        mx = jnp.max(m, axis=1, keepdims=True)
        return jnp.where(m == mx, neg_inf, m)

    m_fin = jax.lax.fori_loop(0, K, body, mag)
    keep = jnp.where((m_fin == neg_inf) & (cols < F),
                     jnp.float32(1.0), jnp.float32(0.0))

    lo = jax.lax.Precision.DEFAULT
    spec_m = spec * jnp.concatenate([keep, keep], axis=1)
    x_filt = jnp.dot(spec_m, inv_ref[...],
                     preferred_element_type=jnp.float32, precision=lo)
    pf = jnp.maximum(
        jnp.dot(x_filt, w1_ref[...], preferred_element_type=jnp.float32,
                precision=lo) + b1_ref[...], 0.0)
    comb = jnp.concatenate([pf, x], axis=1)
    h = jnp.maximum(
        jnp.dot(comb, w2_ref[...], preferred_element_type=jnp.float32,
                precision=lo) + b2_ref[...], 0.0)
    out_mlp = jnp.dot(h, w3_ref[...], preferred_element_type=jnp.float32,
                      precision=lo) + b3_ref[...]
    o_ref[...] = (x - x_filt) + out_mlp


@jax.jit
def kernel(x, W1, b1, W2, b2, W3, b3):
    B, S, C = x.shape
    F = C // 2 + 1
    FP = ((F + 127) // 128) * 128
    K = min(20, F)
    T = B * S
    TB = 1024 if T % 1024 == 0 else T

    fwd_np, inv_np = _dft_mats(C, FP)
    fwd = jnp.asarray(fwd_np)
    fwd_h = fwd.astype(jnp.bfloat16)
    fwd_l = (fwd - fwd_h.astype(jnp.float32)).astype(jnp.bfloat16)
    inv = jnp.asarray(inv_np)

    H1 = W1.shape[1]
    H1P = 128
    w1p = jnp.zeros((C, H1P), jnp.float32).at[:, :H1].set(W1)
    b1p = jnp.zeros((1, H1P), jnp.float32).at[0, :H1].set(b1)
    H2 = W2.shape[1]
    w2p = jnp.zeros((H1P + C, H2), jnp.float32)
    w2p = w2p.at[:H1, :].set(W2[:H1, :]).at[H1P:, :].set(W2[H1:, :])
    b2r = b2.reshape(1, H2)
    b3r = b3.reshape(1, C)

    xt = x.reshape(T, C)
    full = lambda shape: pl.BlockSpec(shape, lambda i: (0, 0))
    out = pl.pallas_call(
        functools.partial(_fan_block, F=F, FP=FP, K=K),
        grid=(T // TB,),
        in_specs=[
            pl.BlockSpec((TB, C), lambda i: (i, 0)),
            full((C, 2 * FP)),
            full((C, 2 * FP)),
            full((2 * FP, C)),
            full((C, H1P)),
            full((1, H1P)),
            full((H1P + C, H2)),
            full((1, H2)),
            full((H2, C)),
            full((1, C)),
        ],
        out_specs=pl.BlockSpec((TB, C), lambda i: (i, 0)),
        out_shape=jax.ShapeDtypeStruct((T, C), jnp.float32),
        compiler_params=pltpu.CompilerParams(
            dimension_semantics=("parallel",)),
    )(xt, fwd_h, fwd_l, inv, w1p, b1p, w2p, b2r, W3, b3r)
    return out.reshape(B, S, C)


# speedup vs baseline: 1.4828x; 1.1967x over previous
import functools

import jax
import jax.numpy as jnp
import numpy as np
from jax.experimental import pallas as pl
from jax.experimental.pallas import tpu as pltpu


def _dft_mats(C: int):
    H = C // 2
    c = np.arange(C)[:, None].astype(np.float64)
    f = np.arange(H)[None, :].astype(np.float64)
    ang = 2.0 * np.pi * c * f / C
    cosm = np.cos(ang)
    sinm = -np.sin(ang)
    sinm[:, 0] = np.cos(np.pi * np.arange(C))
    fwd = np.concatenate([cosm, sinm], axis=1)
    w = np.full((1, H), 2.0 / C)
    w[0, 0] = 1.0 / C
    icos = np.cos(ang) * w
    isin = -np.sin(ang) * w
    isin[:, 0] = np.cos(np.pi * np.arange(C)) / C
    inv = np.concatenate([icos, isin], axis=1).T
    return fwd.astype(np.float32), inv.astype(np.float32)


def _fan_block(x_ref, fwdh_ref, fwdl_ref, inv_ref, w1_ref, b1_ref, w2_ref,
               b2_ref, w3_ref, b3_ref, o_ref, *, H: int, FP: int, K: int):
    x = x_ref[...]
    xh = x.astype(jnp.bfloat16)
    xl = (x - xh.astype(jnp.float32)).astype(jnp.bfloat16)
    fh = fwdh_ref[...]
    spec = (jnp.dot(xh, fh, preferred_element_type=jnp.float32)
            + (jnp.dot(xh, fwdl_ref[...], preferred_element_type=jnp.float32)
               + jnp.dot(xl, fh, preferred_element_type=jnp.float32)))
    re = spec[:, :H]
    imt = spec[:, H:]
    lane = jax.lax.broadcasted_iota(jnp.int32, re.shape, 1)
    im = jnp.where(lane == 0, jnp.float32(0.0), imt)
    mag_lo = re * re + im * im
    pad = jax.lax.broadcasted_iota(jnp.int32, (re.shape[0], FP - H), 1)
    neg_inf = jnp.float32(-jnp.inf)
    nyq = jnp.where(pad == 0, imt[:, 0:1] * imt[:, 0:1], neg_inf)
    mag = jnp.concatenate([mag_lo, nyq], axis=1)

    def body(_, m):
        mx = jnp.max(m, axis=1, keepdims=True)
        return jnp.where(m == mx, neg_inf, m)

    m_fin = jax.lax.fori_loop(0, K, body, mag)
    cols = jax.lax.broadcasted_iota(jnp.int32, mag.shape, 1)
    keep = jnp.where((m_fin == neg_inf) & (cols <= H),
                     jnp.float32(1.0), jnp.float32(0.0))
    keepc = jnp.concatenate(
        [keep[:, :H], keep[:, H:H + 1], keep[:, 1:H]], axis=1)

    lo = jax.lax.Precision.DEFAULT
    spec_m = spec * keepc
    x_filt = jnp.dot(spec_m, inv_ref[...],
                     preferred_element_type=jnp.float32, precision=lo)
    pf = jnp.maximum(
        jnp.dot(x_filt, w1_ref[...], preferred_element_type=jnp.float32,
                precision=lo) + b1_ref[...], 0.0)
    comb = jnp.concatenate([pf, x], axis=1)
    h = jnp.maximum(
        jnp.dot(comb, w2_ref[...], preferred_element_type=jnp.float32,
                precision=lo) + b2_ref[...], 0.0)
    out_mlp = jnp.dot(h, w3_ref[...], preferred_element_type=jnp.float32,
                      precision=lo) + b3_ref[...]
    o_ref[...] = (x - x_filt) + out_mlp


@jax.jit
def kernel(x, W1, b1, W2, b2, W3, b3):
    B, S, C = x.shape
    H = C // 2
    FP = H + 128
    K = min(20, H + 1)
    T = B * S
    TB = 1024 if T % 1024 == 0 else T

    fwd_np, inv_np = _dft_mats(C)
    fwd = jnp.asarray(fwd_np)
    fwd_h = fwd.astype(jnp.bfloat16)
    fwd_l = (fwd - fwd_h.astype(jnp.float32)).astype(jnp.bfloat16)
    inv = jnp.asarray(inv_np)

    H1 = W1.shape[1]
    H1P = 128
    w1p = jnp.zeros((C, H1P), jnp.float32).at[:, :H1].set(W1)
    b1p = jnp.zeros((1, H1P), jnp.float32).at[0, :H1].set(b1)
    H2 = W2.shape[1]
    w2p = jnp.zeros((H1P + C, H2), jnp.float32)
    w2p = w2p.at[:H1, :].set(W2[:H1, :]).at[H1P:, :].set(W2[H1:, :])
    b2r = b2.reshape(1, H2)
    b3r = b3.reshape(1, C)

    xt = x.reshape(T, C)
    full = lambda shape: pl.BlockSpec(shape, lambda i: (0, 0))
    out = pl.pallas_call(
        functools.partial(_fan_block, H=H, FP=FP, K=K),
        grid=(T // TB,),
        in_specs=[
            pl.BlockSpec((TB, C), lambda i: (i, 0)),
            full((C, C)),
            full((C, C)),
            full((C, C)),
            full((C, H1P)),
            full((1, H1P)),
            full((H1P + C, H2)),
            full((1, H2)),
            full((H2, C)),
            full((1, C)),
        ],
        out_specs=pl.BlockSpec((TB, C), lambda i: (i, 0)),
        out_shape=jax.ShapeDtypeStruct((T, C), jnp.float32),
        compiler_params=pltpu.CompilerParams(
            dimension_semantics=("parallel",)),
    )(xt, fwd_h, fwd_l, inv, w1p, b1p, w2p, b2r, W3, b3r)
    return out.reshape(B, S, C)


# speedup vs baseline: 2.3775x; 1.6034x over previous
import functools

import jax
import jax.numpy as jnp
import numpy as np
from jax.experimental import pallas as pl
from jax.experimental.pallas import tpu as pltpu


def _dft_mats(C: int):
    H = C // 2
    c = np.arange(C)[:, None].astype(np.float64)
    f = np.arange(H)[None, :].astype(np.float64)
    ang = 2.0 * np.pi * c * f / C
    cosm = np.cos(ang)
    sinm = -np.sin(ang)
    sinm[:, 0] = np.cos(np.pi * np.arange(C))
    fwd = np.concatenate([cosm, sinm], axis=1)
    w = np.full((1, H), 2.0 / C)
    w[0, 0] = 1.0 / C
    icos = np.cos(ang) * w
    isin = -np.sin(ang) * w
    isin[:, 0] = np.cos(np.pi * np.arange(C)) / C
    inv = np.concatenate([icos, isin], axis=1).T
    return fwd.astype(np.float32), inv.astype(np.float32)


def _fan_block(x_ref, fwdh_ref, fwdl_ref, inv_ref, w1_ref, b1_ref, w2_ref,
               b2_ref, w3_ref, b3_ref, o_ref, *, H: int, FP: int, K: int):
    x = x_ref[...]
    xh = x.astype(jnp.bfloat16)
    xl = (x - xh.astype(jnp.float32)).astype(jnp.bfloat16)
    fh = fwdh_ref[...]
    spec = (jnp.dot(xh, fh, preferred_element_type=jnp.float32)
            + (jnp.dot(xh, fwdl_ref[...], preferred_element_type=jnp.float32)
               + jnp.dot(xl, fh, preferred_element_type=jnp.float32)))
    re = spec[:, :H]
    imt = spec[:, H:]
    lane = jax.lax.broadcasted_iota(jnp.int32, re.shape, 1)
    im = jnp.where(lane == 0, jnp.float32(0.0), imt)
    mag_lo = re * re + im * im
    pad = jax.lax.broadcasted_iota(jnp.int32, (re.shape[0], FP - H), 1)
    neg_inf = jnp.float32(-jnp.inf)
    nyq = jnp.where(pad == 0, imt[:, 0:1] * imt[:, 0:1], neg_inf)
    mag = jnp.concatenate([mag_lo, nyq], axis=1)

    m_fin = mag
    for _ in range(K):
        mx = jnp.max(m_fin, axis=1, keepdims=True)
        m_fin = jnp.where(m_fin == mx, neg_inf, m_fin)
    cols = jax.lax.broadcasted_iota(jnp.int32, mag.shape, 1)
    keep = jnp.where((m_fin == neg_inf) & (cols <= H),
                     jnp.float32(1.0), jnp.float32(0.0))
    keepc = jnp.concatenate(
        [keep[:, :H], keep[:, H:H + 1], keep[:, 1:H]], axis=1)

    lo = jax.lax.Precision.DEFAULT
    spec_m = spec * keepc
    x_filt = jnp.dot(spec_m, inv_ref[...],
                     preferred_element_type=jnp.float32, precision=lo)
    pf = jnp.maximum(
        jnp.dot(x_filt, w1_ref[...], preferred_element_type=jnp.float32,
                precision=lo) + b1_ref[...], 0.0)
    comb = jnp.concatenate([pf, x], axis=1)
    h = jnp.maximum(
        jnp.dot(comb, w2_ref[...], preferred_element_type=jnp.float32,
                precision=lo) + b2_ref[...], 0.0)
    out_mlp = jnp.dot(h, w3_ref[...], preferred_element_type=jnp.float32,
                      precision=lo) + b3_ref[...]
    o_ref[...] = (x - x_filt) + out_mlp


@jax.jit
def kernel(x, W1, b1, W2, b2, W3, b3):
    B, S, C = x.shape
    H = C // 2
    FP = H + 128
    K = min(20, H + 1)
    T = B * S
    TB = 1024 if T % 1024 == 0 else T

    fwd_np, inv_np = _dft_mats(C)
    fwd = jnp.asarray(fwd_np)
    fwd_h = fwd.astype(jnp.bfloat16)
    fwd_l = (fwd - fwd_h.astype(jnp.float32)).astype(jnp.bfloat16)
    inv = jnp.asarray(inv_np)

    H1 = W1.shape[1]
    H1P = 128
    w1p = jnp.zeros((C, H1P), jnp.float32).at[:, :H1].set(W1)
    b1p = jnp.zeros((1, H1P), jnp.float32).at[0, :H1].set(b1)
    H2 = W2.shape[1]
    w2p = jnp.zeros((H1P + C, H2), jnp.float32)
    w2p = w2p.at[:H1, :].set(W2[:H1, :]).at[H1P:, :].set(W2[H1:, :])
    b2r = b2.reshape(1, H2)
    b3r = b3.reshape(1, C)

    xt = x.reshape(T, C)
    full = lambda shape: pl.BlockSpec(shape, lambda i: (0, 0))
    out = pl.pallas_call(
        functools.partial(_fan_block, H=H, FP=FP, K=K),
        grid=(T // TB,),
        in_specs=[
            pl.BlockSpec((TB, C), lambda i: (i, 0)),
            full((C, C)),
            full((C, C)),
            full((C, C)),
            full((C, H1P)),
            full((1, H1P)),
            full((H1P + C, H2)),
            full((1, H2)),
            full((H2, C)),
            full((1, C)),
        ],
        out_specs=pl.BlockSpec((TB, C), lambda i: (i, 0)),
        out_shape=jax.ShapeDtypeStruct((T, C), jnp.float32),
        compiler_params=pltpu.CompilerParams(
            dimension_semantics=("parallel",)),
    )(xt, fwd_h, fwd_l, inv, w1p, b1p, w2p, b2r, W3, b3r)
    return out.reshape(B, S, C)


# speedup vs baseline: 2.5809x; 1.0856x over previous
import functools

import jax
import jax.numpy as jnp
import numpy as np
from jax.experimental import pallas as pl
from jax.experimental.pallas import tpu as pltpu


def _dft_mats(C: int):
    H = C // 2
    c = np.arange(C)[:, None].astype(np.float64)
    f = np.arange(H)[None, :].astype(np.float64)
    ang = 2.0 * np.pi * c * f / C
    cosm = np.cos(ang)
    sinm = -np.sin(ang)
    sinm[:, 0] = np.cos(np.pi * np.arange(C))
    fwd = np.concatenate([cosm, sinm], axis=1)
    w = np.full((1, H), 2.0 / C)
    w[0, 0] = 1.0 / C
    icos = np.cos(ang) * w
    isin = -np.sin(ang) * w
    isin[:, 0] = np.cos(np.pi * np.arange(C)) / C
    inv = np.concatenate([icos, isin], axis=1).T
    return fwd.astype(np.float32), inv.astype(np.float32)


def _spec_mag(x, fwdh, fwdl, *, H, FP):
    xh = x.astype(jnp.bfloat16)
    xl = (x - xh.astype(jnp.float32)).astype(jnp.bfloat16)
    spec = (jnp.dot(xh, fwdh, preferred_element_type=jnp.float32)
            + (jnp.dot(xh, fwdl, preferred_element_type=jnp.float32)
               + jnp.dot(xl, fwdh, preferred_element_type=jnp.float32)))
    re = spec[:, :H]
    imt = spec[:, H:]
    lane = jax.lax.broadcasted_iota(jnp.int32, re.shape, 1)
    im = jnp.where(lane == 0, jnp.float32(0.0), imt)
    mag_lo = re * re + im * im
    pad = jax.lax.broadcasted_iota(jnp.int32, (re.shape[0], FP - H), 1)
    neg_inf = jnp.float32(-jnp.inf)
    nyq = jnp.where(pad == 0, imt[:, 0:1] * imt[:, 0:1], neg_inf)
    return spec, jnp.concatenate([mag_lo, nyq], axis=1)


def _tail(x, spec, m_fin, inv, w1, b1, w2, b2, w3, b3, *, H):
    neg_inf = jnp.float32(-jnp.inf)
    cols = jax.lax.broadcasted_iota(jnp.int32, m_fin.shape, 1)
    keep = jnp.where((m_fin == neg_inf) & (cols <= H),
                     jnp.float32(1.0), jnp.float32(0.0))
    keepc = jnp.concatenate(
        [keep[:, :H], keep[:, H:H + 1], keep[:, 1:H]], axis=1)

    lo = jax.lax.Precision.DEFAULT
    spec_m = spec * keepc
    x_filt = jnp.dot(spec_m, inv,
                     preferred_element_type=jnp.float32, precision=lo)
    pf = jnp.maximum(
        jnp.dot(x_filt, w1, preferred_element_type=jnp.float32,
                precision=lo) + b1, 0.0)
    comb = jnp.concatenate([pf, x], axis=1)
    h = jnp.maximum(
        jnp.dot(comb, w2, preferred_element_type=jnp.float32,
                precision=lo) + b2, 0.0)
    out_mlp = jnp.dot(h, w3, preferred_element_type=jnp.float32,
                      precision=lo) + b3
    return (x - x_filt) + out_mlp


def _fan_block(x_ref, fwdh_ref, fwdl_ref, inv_ref, w1_ref, b1_ref, w2_ref,
               b2_ref, w3_ref, b3_ref, o_ref, *, H: int, FP: int, K: int):
    TB = x_ref.shape[0]
    TBH = TB // 2
    neg_inf = jnp.float32(-jnp.inf)
    args = (inv_ref[...], w1_ref[...], b1_ref[...], w2_ref[...],
            b2_ref[...], w3_ref[...], b3_ref[...])
    xa = x_ref[:TBH, :]
    xb = x_ref[TBH:, :]
    fh = fwdh_ref[...]
    fl = fwdl_ref[...]
    spec_a, ma = _spec_mag(xa, fh, fl, H=H, FP=FP)
    spec_b, mb = _spec_mag(xb, fh, fl, H=H, FP=FP)
    for _ in range(K):
        mxa = jnp.max(ma, axis=1, keepdims=True)
        mxb = jnp.max(mb, axis=1, keepdims=True)
        ma = jnp.where(ma == mxa, neg_inf, ma)
        mb = jnp.where(mb == mxb, neg_inf, mb)
    o_ref[:TBH, :] = _tail(xa, spec_a, ma, *args, H=H)
    o_ref[TBH:, :] = _tail(xb, spec_b, mb, *args, H=H)


@jax.jit
def kernel(x, W1, b1, W2, b2, W3, b3):
    B, S, C = x.shape
    H = C // 2
    FP = H + 128
    K = min(20, H + 1)
    T = B * S
    TB = 1024 if T % 1024 == 0 else T

    fwd_np, inv_np = _dft_mats(C)
    fwd = jnp.asarray(fwd_np)
    fwd_h = fwd.astype(jnp.bfloat16)
    fwd_l = (fwd - fwd_h.astype(jnp.float32)).astype(jnp.bfloat16)
    inv = jnp.asarray(inv_np)

    H1 = W1.shape[1]
    H1P = 128
    w1p = jnp.zeros((C, H1P), jnp.float32).at[:, :H1].set(W1)
    b1p = jnp.zeros((1, H1P), jnp.float32).at[0, :H1].set(b1)
    H2 = W2.shape[1]
    w2p = jnp.zeros((H1P + C, H2), jnp.float32)
    w2p = w2p.at[:H1, :].set(W2[:H1, :]).at[H1P:, :].set(W2[H1:, :])
    b2r = b2.reshape(1, H2)
    b3r = b3.reshape(1, C)

    xt = x.reshape(T, C)
    full = lambda shape: pl.BlockSpec(shape, lambda i: (0, 0))
    out = pl.pallas_call(
        functools.partial(_fan_block, H=H, FP=FP, K=K),
        grid=(T // TB,),
        in_specs=[
            pl.BlockSpec((TB, C), lambda i: (i, 0)),
            full((C, C)),
            full((C, C)),
            full((C, C)),
            full((C, H1P)),
            full((1, H1P)),
            full((H1P + C, H2)),
            full((1, H2)),
            full((H2, C)),
            full((1, C)),
        ],
        out_specs=pl.BlockSpec((TB, C), lambda i: (i, 0)),
        out_shape=jax.ShapeDtypeStruct((T, C), jnp.float32),
        compiler_params=pltpu.CompilerParams(
            dimension_semantics=("parallel",)),
    )(xt, fwd_h, fwd_l, inv, w1p, b1p, w2p, b2r, W3, b3r)
    return out.reshape(B, S, C)


# speedup vs baseline: 2.6288x; 1.0185x over previous
import functools

import jax
import jax.numpy as jnp
import numpy as np
from jax.experimental import pallas as pl
from jax.experimental.pallas import tpu as pltpu


def _dft_mats(C: int):
    H = C // 2
    c = np.arange(C)[:, None].astype(np.float64)
    f = np.arange(H)[None, :].astype(np.float64)
    ang = 2.0 * np.pi * c * f / C
    cosm = np.cos(ang)
    sinm = -np.sin(ang)
    sinm[:, 0] = np.cos(np.pi * np.arange(C))
    fwd = np.concatenate([cosm, sinm], axis=1)
    w = np.full((1, H), 2.0 / C)
    w[0, 0] = 1.0 / C
    icos = np.cos(ang) * w
    isin = -np.sin(ang) * w
    isin[:, 0] = np.cos(np.pi * np.arange(C)) / C
    inv = np.concatenate([icos, isin], axis=1).T
    return fwd.astype(np.float32), inv.astype(np.float32)


def _spec_mag(x, fwdh, fwdl, *, H, FP):
    xh = x.astype(jnp.bfloat16)
    xl = (x - xh.astype(jnp.float32)).astype(jnp.bfloat16)
    spec = (jnp.dot(xh, fwdh, preferred_element_type=jnp.float32)
            + (jnp.dot(xh, fwdl, preferred_element_type=jnp.float32)
               + jnp.dot(xl, fwdh, preferred_element_type=jnp.float32)))
    re = spec[:, :H]
    imt = spec[:, H:]
    lane = jax.lax.broadcasted_iota(jnp.int32, re.shape, 1)
    im = jnp.where(lane == 0, jnp.float32(0.0), imt)
    mag_lo = re * re + im * im
    pad = jax.lax.broadcasted_iota(jnp.int32, (re.shape[0], FP - H), 1)
    neg_inf = jnp.float32(-jnp.inf)
    nyq = jnp.where(pad == 0, imt[:, 0:1] * imt[:, 0:1], neg_inf)
    return spec, jnp.concatenate([mag_lo, nyq], axis=1)


def _tail(x, spec, mag, thr, inv, w1, b1, w2, b2, w3, b3, *, H):
    cols = jax.lax.broadcasted_iota(jnp.int32, mag.shape, 1)
    keep = jnp.where((mag >= thr) & (cols <= H),
                     jnp.float32(1.0), jnp.float32(0.0))
    keepc = jnp.concatenate(
        [keep[:, :H], keep[:, H:H + 1], keep[:, 1:H]], axis=1)

    lo = jax.lax.Precision.DEFAULT
    spec_m = spec * keepc
    x_filt = jnp.dot(spec_m, inv,
                     preferred_element_type=jnp.float32, precision=lo)
    pf = jnp.maximum(
        jnp.dot(x_filt, w1, preferred_element_type=jnp.float32,
                precision=lo) + b1, 0.0)
    comb = jnp.concatenate([pf, x], axis=1)
    h = jnp.maximum(
        jnp.dot(comb, w2, preferred_element_type=jnp.float32,
                precision=lo) + b2, 0.0)
    out_mlp = jnp.dot(h, w3, preferred_element_type=jnp.float32,
                      precision=lo) + b3
    return (x - x_filt) + out_mlp


def _fan_block(x_ref, fwdh_ref, fwdl_ref, inv_ref, w1_ref, b1_ref, w2_ref,
               b2_ref, w3_ref, b3_ref, o_ref, *, H: int, FP: int, K: int):
    TB = x_ref.shape[0]
    TBH = TB // 2
    neg_inf = jnp.float32(-jnp.inf)
    args = (inv_ref[...], w1_ref[...], b1_ref[...], w2_ref[...],
            b2_ref[...], w3_ref[...], b3_ref[...])
    fh = fwdh_ref[...]
    fl = fwdl_ref[...]
    NW = 4
    TQ = TB // NW
    xs = [x_ref[i * TQ:(i + 1) * TQ, :] for i in range(NW)]
    sm = [_spec_mag(xq, fh, fl, H=H, FP=FP) for xq in xs]
    specs = [s for s, _ in sm]
    ms = [m for _, m in sm]
    mxs = [jnp.max(m, axis=1, keepdims=True) for m in ms]
    for _ in range(K - 1):
        mxs = [jnp.max(jnp.where(m >= mx, neg_inf, m), axis=1, keepdims=True)
               for m, mx in zip(ms, mxs)]
    for i in range(NW):
        o_ref[i * TQ:(i + 1) * TQ, :] = _tail(xs[i], specs[i], ms[i], mxs[i],
                                              *args, H=H)


@jax.jit
def kernel(x, W1, b1, W2, b2, W3, b3):
    B, S, C = x.shape
    H = C // 2
    FP = H + 128
    K = min(20, H + 1)
    T = B * S
    TB = 1024 if T % 1024 == 0 else T

    fwd_np, inv_np = _dft_mats(C)
    fwd = jnp.asarray(fwd_np)
    fwd_h = fwd.astype(jnp.bfloat16)
    fwd_l = (fwd - fwd_h.astype(jnp.float32)).astype(jnp.bfloat16)
    inv = jnp.asarray(inv_np)

    H1 = W1.shape[1]
    H1P = 128
    w1p = jnp.zeros((C, H1P), jnp.float32).at[:, :H1].set(W1)
    b1p = jnp.zeros((1, H1P), jnp.float32).at[0, :H1].set(b1)
    H2 = W2.shape[1]
    w2p = jnp.zeros((H1P + C, H2), jnp.float32)
    w2p = w2p.at[:H1, :].set(W2[:H1, :]).at[H1P:, :].set(W2[H1:, :])
    b2r = b2.reshape(1, H2)
    b3r = b3.reshape(1, C)

    xt = x.reshape(T, C)
    full = lambda shape: pl.BlockSpec(shape, lambda i: (0, 0))
    out = pl.pallas_call(
        functools.partial(_fan_block, H=H, FP=FP, K=K),
        grid=(T // TB,),
        in_specs=[
            pl.BlockSpec((TB, C), lambda i: (i, 0)),
            full((C, C)),
            full((C, C)),
            full((C, C)),
            full((C, H1P)),
            full((1, H1P)),
            full((H1P + C, H2)),
            full((1, H2)),
            full((H2, C)),
            full((1, C)),
        ],
        out_specs=pl.BlockSpec((TB, C), lambda i: (i, 0)),
        out_shape=jax.ShapeDtypeStruct((T, C), jnp.float32),
        compiler_params=pltpu.CompilerParams(
            dimension_semantics=("parallel",)),
    )(xt, fwd_h, fwd_l, inv, w1p, b1p, w2p, b2r, W3, b3r)
    return out.reshape(B, S, C)
